# baseline (speedup 1.0000x reference)
import jax
import jax.numpy as jnp
from jax.experimental import pallas as pl
from jax.experimental.pallas import tpu as pltpu

DIM = 768
D_STATE = 16
D_CONV = 4
E = 8
D_INNER = 2 * DIM
DT_RANK = (DIM + 15) // 16
HID = 4 * DIM
L = 2048
SB = 256
NSB = L // SB
KS = 8

_f32 = jnp.float32
_bf16 = jnp.bfloat16


def _rmsn(v):
    return v * jax.lax.rsqrt(jnp.mean(v * v, axis=-1, keepdims=True) + 1e-8)


def _silu(v):
    return v * jax.nn.sigmoid(v)


def _km_body(x_ref, win_ref, wc_ref, cb_ref, wx_ref, wdt_ref, bdt_ref,
             dp_ref, alogt_ref, wout_ref, wg_ref,
             h2_ref, g_ref,
             halo_ref, h_ref, dts_ref, us_ref, bs_ref, cs_ref, ys_ref):
    i = pl.program_id(0)
    xb = x_ref[...]
    h1 = _rmsn(xb)
    xz = jnp.dot(h1.astype(_bf16), win_ref[...], preferred_element_type=_f32)
    xi_raw = xz[:, :D_INNER]
    z = xz[:, D_INNER:]

    halo = jnp.where(i > 0, halo_ref[...], 0.0)
    ext = jnp.concatenate([halo[5:], xi_raw], axis=0)
    halo_ref[...] = xi_raw[SB - 8:]
    wc = wc_ref[...]
    xc = (ext[0:SB] * wc[0:1] + ext[1:SB + 1] * wc[1:2]
          + ext[2:SB + 2] * wc[2:3] + ext[3:SB + 3] * wc[3:4]) + cb_ref[...]
    xi = _silu(xc)
    x_dbl = jnp.dot(xi, wx_ref[...], preferred_element_type=_f32)
    dt = jax.nn.softplus(
        jnp.dot(x_dbl[:, :DT_RANK], wdt_ref[...], preferred_element_type=_f32)
        + bdt_ref[...])
    dts_ref[...] = dt
    us_ref[...] = dt * xi
    bs_ref[...] = x_dbl[:, DT_RANK:DT_RANK + D_STATE]
    cs_ref[...] = x_dbl[:, DT_RANK + D_STATE:DT_RANK + 2 * D_STATE]

    @pl.when(i == 0)
    def _():
        h_ref[...] = jnp.zeros_like(h_ref)

    at = -jnp.exp(alogt_ref[...])
    at_tile = jnp.concatenate([at] * KS, axis=0)
    r1 = jax.lax.broadcasted_iota(jnp.int32, (KS * D_STATE, KS), 0)
    c1 = jax.lax.broadcasted_iota(jnp.int32, (KS * D_STATE, KS), 1)
    rsel = (r1 // D_STATE == c1).astype(_bf16)
    r2 = jax.lax.broadcasted_iota(jnp.int32, (KS * D_STATE, D_STATE), 0)
    c2 = jax.lax.broadcasted_iota(jnp.int32, (KS * D_STATE, D_STATE), 1)
    nmask = (r2 % D_STATE == c2).astype(_f32)
    ones16 = jnp.ones((D_STATE, 1), _bf16)
    r3 = jax.lax.broadcasted_iota(jnp.int32, (D_STATE, KS * D_STATE), 0)
    c3 = jax.lax.broadcasted_iota(jnp.int32, (D_STATE, KS * D_STATE), 1)
    tile16 = (c3 % D_STATE == r3).astype(_bf16)
    r4 = jax.lax.broadcasted_iota(jnp.int32, (KS, KS * D_STATE), 0)
    c4 = jax.lax.broadcasted_iota(jnp.int32, (KS, KS * D_STATE), 1)
    smask = (c4 // D_STATE == r4).astype(_f32)

    def outer(g, h):
        s = g * KS
        dt_blk = dts_ref[pl.ds(s, KS), :]
        u_blk = us_ref[pl.ds(s, KS), :]
        b_blk = bs_ref[pl.ds(s, KS), :]
        c_blk = cs_ref[pl.ds(s, KS), :]
        dtrep = jax.lax.dot_general(rsel, dt_blk.astype(_bf16),
                                    (((1,), (0,)), ((), ())),
                                    preferred_element_type=_f32)
        da = jnp.exp(dtrep * at_tile)
        urep = jax.lax.dot_general(rsel, u_blk.astype(_bf16),
                                   (((1,), (0,)), ((), ())),
                                   preferred_element_type=_f32)
        brep = jax.lax.dot_general(rsel, b_blk.astype(_bf16),
                                   (((1,), (0,)), ((), ())),
                                   preferred_element_type=_f32)
        bcol = jnp.dot((brep * nmask).astype(_bf16), ones16,
                       preferred_element_type=_f32)
        dbx = bcol * urep
        ctile = jnp.dot(c_blk.astype(_bf16), tile16,
                        preferred_element_type=_f32)
        sm = (ctile * smask).astype(_bf16)
        hs = []
        for t in range(KS):
            h = (h * da[t * D_STATE:(t + 1) * D_STATE]
                 + dbx[t * D_STATE:(t + 1) * D_STATE])
            hs.append(h)
        hstk = jnp.concatenate(hs, axis=0).astype(_bf16)
        y_blk = jnp.dot(sm, hstk, preferred_element_type=_f32)
        ys_ref[pl.ds(s, KS), :] = y_blk
        return h

    h = jax.lax.fori_loop(0, SB // KS, outer, h_ref[...])
    h_ref[...] = h

    yg = (ys_ref[...] + xi * dp_ref[...]) * _silu(z)
    y2 = jnp.dot(yg.astype(_bf16), wout_ref[...], preferred_element_type=_f32)
    h2 = _rmsn(y2 + h1)
    h2_ref[...] = h2.astype(_bf16)
    logits = jnp.dot(h2, wg_ref[...], preferred_element_type=_f32)
    ii = jax.lax.broadcasted_iota(jnp.int32, (SB, E), 1)
    v1 = jnp.max(logits, axis=1, keepdims=True)
    i1 = jnp.min(jnp.where(logits == v1, ii, E), axis=1, keepdims=True)
    l2 = jnp.where(ii == i1, -1e30, logits)
    v2 = jnp.max(l2, axis=1, keepdims=True)
    i2 = jnp.min(jnp.where(l2 == v2, ii, E), axis=1, keepdims=True)
    w1 = jax.nn.sigmoid(v1 - v2)
    g_ref[...] = (jnp.where(ii == i1, w1, 0.0)
                  + jnp.where(ii == i2, 1.0 - w1, 0.0))


def _ke_body(h2_ref, g_ref, w1_ref, b1_ref, w2_ref, b2_ref, x_ref, out_ref):
    e = pl.program_id(0)

    @pl.when(e == 0)
    def _():
        out_ref[...] = x_ref[...]

    for j in range(NSB):
        h2 = h2_ref[j * SB:(j + 1) * SB, :]
        m1 = jnp.dot(h2, w1_ref[0], preferred_element_type=_f32) + b1_ref[0]
        a = jax.nn.gelu(m1)
        eo = (jnp.dot(a.astype(_bf16), w2_ref[0], preferred_element_type=_f32)
              + b2_ref[0])
        g = g_ref[j * SB:(j + 1) * SB, :]
        ge = jnp.sum(
            jnp.where(jax.lax.broadcasted_iota(jnp.int32, (SB, E), 1) == e,
                      g, 0.0),
            axis=1, keepdims=True)
        out_ref[j * SB:(j + 1) * SB, :] += ge * eo


def kernel(x, W_in, conv_w, conv_b, W_xproj, W_dt, b_dt, A_log, Dp, W_out,
           W_gate, W1, b1, W2, b2):
    x2 = x[0]
    win_b = W_in.astype(_bf16)
    wout_b = W_out.astype(_bf16)
    w1_b = W1.astype(_bf16)
    w2_b = W2.astype(_bf16)
    wc = jnp.transpose(conv_w[:, 0, :], (1, 0))
    alogt = jnp.transpose(A_log, (1, 0))

    h2b, gates = pl.pallas_call(
        _km_body,
        grid=(NSB,),
        in_specs=[
            pl.BlockSpec((SB, DIM), lambda i: (i, 0)),
            pl.BlockSpec((DIM, 2 * D_INNER), lambda i: (0, 0)),
            pl.BlockSpec((4, D_INNER), lambda i: (0, 0)),
            pl.BlockSpec((1, D_INNER), lambda i: (0, 0)),
            pl.BlockSpec((D_INNER, DT_RANK + 2 * D_STATE), lambda i: (0, 0)),
            pl.BlockSpec((DT_RANK, D_INNER), lambda i: (0, 0)),
            pl.BlockSpec((1, D_INNER), lambda i: (0, 0)),
            pl.BlockSpec((1, D_INNER), lambda i: (0, 0)),
            pl.BlockSpec((D_STATE, D_INNER), lambda i: (0, 0)),
            pl.BlockSpec((D_INNER, DIM), lambda i: (0, 0)),
            pl.BlockSpec((DIM, E), lambda i: (0, 0)),
        ],
        out_specs=[
            pl.BlockSpec((SB, DIM), lambda i: (i, 0)),
            pl.BlockSpec((SB, E), lambda i: (i, 0)),
        ],
        out_shape=[
            jax.ShapeDtypeStruct((L, DIM), _bf16),
            jax.ShapeDtypeStruct((L, E), _f32),
        ],
        scratch_shapes=[
            pltpu.VMEM((8, D_INNER), _f32),
            pltpu.VMEM((D_STATE, D_INNER), _f32),
            pltpu.VMEM((SB, D_INNER), _f32),
            pltpu.VMEM((SB, D_INNER), _f32),
            pltpu.VMEM((SB, D_STATE), _f32),
            pltpu.VMEM((SB, D_STATE), _f32),
            pltpu.VMEM((SB, D_INNER), _f32),
        ],
    )(x2, win_b, wc, conv_b[None, :], W_xproj, W_dt, b_dt[None, :],
      Dp[None, :], alogt, wout_b, W_gate)

    out = pl.pallas_call(
        _ke_body,
        grid=(E,),
        in_specs=[
            pl.BlockSpec((L, DIM), lambda e: (0, 0)),
            pl.BlockSpec((L, E), lambda e: (0, 0)),
            pl.BlockSpec((1, DIM, HID), lambda e: (e, 0, 0)),
            pl.BlockSpec((1, 1, HID), lambda e: (e, 0, 0)),
            pl.BlockSpec((1, HID, DIM), lambda e: (e, 0, 0)),
            pl.BlockSpec((1, 1, DIM), lambda e: (e, 0, 0)),
            pl.BlockSpec((L, DIM), lambda e: (0, 0)),
        ],
        out_specs=pl.BlockSpec((L, DIM), lambda e: (0, 0)),
        out_shape=jax.ShapeDtypeStruct((L, DIM), _f32),
    )(h2b, gates, w1_b, b1[:, None, :], w2_b, b2[:, None, :], x2)

    return out[None]


# speedup vs baseline: 1.0931x; 1.0931x over previous
import jax
import jax.numpy as jnp
from jax.experimental import pallas as pl
from jax.experimental.pallas import tpu as pltpu

DIM = 768
D_STATE = 16
D_CONV = 4
E = 8
D_INNER = 2 * DIM
DT_RANK = (DIM + 15) // 16
HID = 4 * DIM
L = 2048
SB = 256
NSB = L // SB
KS = 16

_f32 = jnp.float32
_bf16 = jnp.bfloat16


def _rmsn(v):
    return v * jax.lax.rsqrt(jnp.mean(v * v, axis=-1, keepdims=True) + 1e-8)


def _silu(v):
    return v * jax.nn.sigmoid(v)


def _km_body(x_ref, win_ref, wc_ref, cb_ref, wx_ref, wdt_ref, bdt_ref,
             dp_ref, alogt_ref, wout_ref, wg_ref,
             h2_ref, g_ref,
             halo_ref, h_ref, dts_ref, us_ref, bs_ref, cs_ref, ys_ref):
    i = pl.program_id(0)
    xb = x_ref[...]
    h1 = _rmsn(xb)
    xz = jnp.dot(h1.astype(_bf16), win_ref[...], preferred_element_type=_f32)
    xi_raw = xz[:, :D_INNER]
    z = xz[:, D_INNER:]

    halo = jnp.where(i > 0, halo_ref[...], 0.0)
    ext8 = jnp.concatenate([halo, xi_raw], axis=0).astype(_bf16)
    halo_ref[...] = xi_raw[SB - 8:]
    wc = wc_ref[...]
    rs = jax.lax.broadcasted_iota(jnp.int32, (SB, SB + 8), 0)
    cs = jax.lax.broadcasted_iota(jnp.int32, (SB, SB + 8), 1)
    xc = cb_ref[...] * jnp.ones((SB, 1), _f32)
    for j in range(D_CONV):
        shj = (cs == rs + 5 + j).astype(_bf16)
        xc = xc + wc[j:j + 1] * jax.lax.dot_general(
            shj, ext8, (((1,), (0,)), ((), ())), preferred_element_type=_f32)
    xi = _silu(xc)
    x_dbl = jnp.dot(xi, wx_ref[...], preferred_element_type=_f32)
    dt = jax.nn.softplus(
        jnp.dot(x_dbl[:, :DT_RANK], wdt_ref[...], preferred_element_type=_f32)
        + bdt_ref[...])
    dts_ref[...] = dt
    us_ref[...] = dt * xi
    bs_ref[...] = x_dbl[:, DT_RANK:DT_RANK + D_STATE]
    cs_ref[...] = x_dbl[:, DT_RANK + D_STATE:DT_RANK + 2 * D_STATE]

    @pl.when(i == 0)
    def _():
        h_ref[...] = jnp.zeros_like(h_ref)

    at = -jnp.exp(alogt_ref[...])
    at_tile = jnp.concatenate([at] * KS, axis=0)
    r1 = jax.lax.broadcasted_iota(jnp.int32, (KS * D_STATE, KS), 0)
    c1 = jax.lax.broadcasted_iota(jnp.int32, (KS * D_STATE, KS), 1)
    rsel = (r1 // D_STATE == c1).astype(_bf16)
    r2 = jax.lax.broadcasted_iota(jnp.int32, (KS * D_STATE, D_STATE), 0)
    c2 = jax.lax.broadcasted_iota(jnp.int32, (KS * D_STATE, D_STATE), 1)
    nmask = (r2 % D_STATE == c2).astype(_f32)
    ones16 = jnp.ones((D_STATE, 1), _bf16)
    r3 = jax.lax.broadcasted_iota(jnp.int32, (D_STATE, KS * D_STATE), 0)
    c3 = jax.lax.broadcasted_iota(jnp.int32, (D_STATE, KS * D_STATE), 1)
    tile16 = (c3 % D_STATE == r3).astype(_bf16)
    r4 = jax.lax.broadcasted_iota(jnp.int32, (KS, KS * D_STATE), 0)
    c4 = jax.lax.broadcasted_iota(jnp.int32, (KS, KS * D_STATE), 1)
    smask = (c4 // D_STATE == r4).astype(_f32)

    def outer(g, h):
        s = g * KS
        dt_blk = dts_ref[pl.ds(s, KS), :]
        u_blk = us_ref[pl.ds(s, KS), :]
        b_blk = bs_ref[pl.ds(s, KS), :]
        c_blk = cs_ref[pl.ds(s, KS), :]
        dtrep = jax.lax.dot_general(rsel, dt_blk.astype(_bf16),
                                    (((1,), (0,)), ((), ())),
                                    preferred_element_type=_f32)
        da = jnp.exp(dtrep * at_tile)
        urep = jax.lax.dot_general(rsel, u_blk.astype(_bf16),
                                   (((1,), (0,)), ((), ())),
                                   preferred_element_type=_f32)
        brep = jax.lax.dot_general(rsel, b_blk.astype(_bf16),
                                   (((1,), (0,)), ((), ())),
                                   preferred_element_type=_f32)
        bcol = jnp.dot((brep * nmask).astype(_bf16), ones16,
                       preferred_element_type=_f32)
        dbx = bcol * urep
        ctile = jnp.dot(c_blk.astype(_bf16), tile16,
                        preferred_element_type=_f32)
        sm = (ctile * smask).astype(_bf16)
        hs = []
        for t in range(KS):
            h = (h * da[t * D_STATE:(t + 1) * D_STATE]
                 + dbx[t * D_STATE:(t + 1) * D_STATE])
            hs.append(h)
        hstk = jnp.concatenate(hs, axis=0).astype(_bf16)
        y_blk = jnp.dot(sm, hstk, preferred_element_type=_f32)
        ys_ref[pl.ds(s, KS), :] = y_blk
        return h

    h = jax.lax.fori_loop(0, SB // KS, outer, h_ref[...])
    h_ref[...] = h

    yg = (ys_ref[...] + xi * dp_ref[...]) * _silu(z)
    y2 = jnp.dot(yg.astype(_bf16), wout_ref[...], preferred_element_type=_f32)
    h2 = _rmsn(y2 + h1)
    h2_ref[...] = h2.astype(_bf16)
    logits = jnp.dot(h2, wg_ref[...], preferred_element_type=_f32)
    ii = jax.lax.broadcasted_iota(jnp.int32, (SB, E), 1)
    v1 = jnp.max(logits, axis=1, keepdims=True)
    i1 = jnp.min(jnp.where(logits == v1, ii, E), axis=1, keepdims=True)
    l2 = jnp.where(ii == i1, -1e30, logits)
    v2 = jnp.max(l2, axis=1, keepdims=True)
    i2 = jnp.min(jnp.where(l2 == v2, ii, E), axis=1, keepdims=True)
    w1 = jax.nn.sigmoid(v1 - v2)
    g_ref[...] = (jnp.where(ii == i1, w1, 0.0)
                  + jnp.where(ii == i2, 1.0 - w1, 0.0))


def _ke_body(h2_ref, g_ref, w1_ref, b1_ref, w2_ref, b2_ref, x_ref, out_ref):
    e = pl.program_id(0)

    @pl.when(e == 0)
    def _():
        out_ref[...] = x_ref[...]

    for j in range(NSB):
        h2 = h2_ref[j * SB:(j + 1) * SB, :]
        m1 = jnp.dot(h2, w1_ref[0], preferred_element_type=_f32) + b1_ref[0]
        a = jax.nn.gelu(m1)
        eo = (jnp.dot(a.astype(_bf16), w2_ref[0], preferred_element_type=_f32)
              + b2_ref[0])
        g = g_ref[j * SB:(j + 1) * SB, :]
        ge = jnp.sum(
            jnp.where(jax.lax.broadcasted_iota(jnp.int32, (SB, E), 1) == e,
                      g, 0.0),
            axis=1, keepdims=True)
        out_ref[j * SB:(j + 1) * SB, :] += ge * eo


def kernel(x, W_in, conv_w, conv_b, W_xproj, W_dt, b_dt, A_log, Dp, W_out,
           W_gate, W1, b1, W2, b2):
    x2 = x[0]
    win_b = W_in.astype(_bf16)
    wout_b = W_out.astype(_bf16)
    w1_b = W1.astype(_bf16)
    w2_b = W2.astype(_bf16)
    wc = jnp.transpose(conv_w[:, 0, :], (1, 0))
    alogt = jnp.transpose(A_log, (1, 0))

    h2b, gates = pl.pallas_call(
        _km_body,
        grid=(NSB,),
        in_specs=[
            pl.BlockSpec((SB, DIM), lambda i: (i, 0)),
            pl.BlockSpec((DIM, 2 * D_INNER), lambda i: (0, 0)),
            pl.BlockSpec((4, D_INNER), lambda i: (0, 0)),
            pl.BlockSpec((1, D_INNER), lambda i: (0, 0)),
            pl.BlockSpec((D_INNER, DT_RANK + 2 * D_STATE), lambda i: (0, 0)),
            pl.BlockSpec((DT_RANK, D_INNER), lambda i: (0, 0)),
            pl.BlockSpec((1, D_INNER), lambda i: (0, 0)),
            pl.BlockSpec((1, D_INNER), lambda i: (0, 0)),
            pl.BlockSpec((D_STATE, D_INNER), lambda i: (0, 0)),
            pl.BlockSpec((D_INNER, DIM), lambda i: (0, 0)),
            pl.BlockSpec((DIM, E), lambda i: (0, 0)),
        ],
        out_specs=[
            pl.BlockSpec((SB, DIM), lambda i: (i, 0)),
            pl.BlockSpec((SB, E), lambda i: (i, 0)),
        ],
        out_shape=[
            jax.ShapeDtypeStruct((L, DIM), _bf16),
            jax.ShapeDtypeStruct((L, E), _f32),
        ],
        scratch_shapes=[
            pltpu.VMEM((8, D_INNER), _f32),
            pltpu.VMEM((D_STATE, D_INNER), _f32),
            pltpu.VMEM((SB, D_INNER), _f32),
            pltpu.VMEM((SB, D_INNER), _f32),
            pltpu.VMEM((SB, D_STATE), _f32),
            pltpu.VMEM((SB, D_STATE), _f32),
            pltpu.VMEM((SB, D_INNER), _f32),
        ],
    )(x2, win_b, wc, conv_b[None, :], W_xproj, W_dt, b_dt[None, :],
      Dp[None, :], alogt, wout_b, W_gate)

    out = pl.pallas_call(
        _ke_body,
        grid=(E,),
        in_specs=[
            pl.BlockSpec((L, DIM), lambda e: (0, 0)),
            pl.BlockSpec((L, E), lambda e: (0, 0)),
            pl.BlockSpec((1, DIM, HID), lambda e: (e, 0, 0)),
            pl.BlockSpec((1, 1, HID), lambda e: (e, 0, 0)),
            pl.BlockSpec((1, HID, DIM), lambda e: (e, 0, 0)),
            pl.BlockSpec((1, 1, DIM), lambda e: (e, 0, 0)),
            pl.BlockSpec((L, DIM), lambda e: (0, 0)),
        ],
        out_specs=pl.BlockSpec((L, DIM), lambda e: (0, 0)),
        out_shape=jax.ShapeDtypeStruct((L, DIM), _f32),
    )(h2b, gates, w1_b, b1[:, None, :], w2_b, b2[:, None, :], x2)

    return out[None]
